# baseline (speedup 1.0000x reference)
import functools

import jax
import jax.numpy as jnp
from jax import lax
from jax.experimental import pallas as pl
from jax.experimental.pallas import tpu as pltpu
from jax.experimental.pallas import tpu_sc as plsc

NC = 2
NS = 16
NW = NC * NS
GRP = 8
D = 32
CH = 128


@functools.lru_cache(maxsize=None)
def _make_gather(B):
    b_per_w = B // NW
    n_chunks = b_per_w // CH
    mesh = plsc.VectorSubcoreMesh(
        core_axis_name="c", subcore_axis_name="s", num_cores=NC, num_subcores=NS
    )

    @functools.partial(
        pl.kernel,
        out_type=[jax.ShapeDtypeStruct((B, D), jnp.float32) for _ in range(4)],
        mesh=mesh,
        scratch_types=[
            pltpu.VMEM((b_per_w + 16,), jnp.int32),
            pltpu.VMEM((b_per_w + 16,), jnp.int32),
            pltpu.VMEM((CH, D), jnp.float32),
            pltpu.VMEM((CH, D), jnp.float32),
            pltpu.VMEM((CH, D), jnp.float32),
            pltpu.VMEM((CH, D), jnp.float32),
            pltpu.SemaphoreType.DMA,
            pltpu.SemaphoreType.DMA,
            pltpu.SemaphoreType.DMA,
            pltpu.SemaphoreType.DMA,
        ],
    )
    def gather_kernel(uids_hbm, iids_hbm, t0_hbm, t1_hbm, t2_hbm, t3_hbm,
                      out0, out1, out2, out3,
                      uidx_v, iidx_v, b0, b1, b2, b3, s0, s1, s2, s3):
        wid = lax.axis_index("s") * NC + lax.axis_index("c")
        base = wid * b_per_w
        pltpu.sync_copy(uids_hbm.at[pl.ds(base, b_per_w)],
                        uidx_v.at[pl.ds(0, b_per_w)])
        pltpu.sync_copy(iids_hbm.at[pl.ds(base, b_per_w)],
                        iidx_v.at[pl.ds(0, b_per_w)])

        def chunk(k2, _):
            def grp_u(k, _):
                v = uidx_v[pl.ds(k2 * CH + k * GRP, 16)]
                for l in range(GRP):
                    dst = pl.ds(k * GRP + l, 1)
                    pltpu.async_copy(t0_hbm.at[pl.ds(v[l], 1)],
                                     b0.at[dst], s0)
                    pltpu.async_copy(t2_hbm.at[pl.ds(v[l], 1)],
                                     b2.at[dst], s2)
                return 0

            def grp_i(k, _):
                v = iidx_v[pl.ds(k2 * CH + k * GRP, 16)]
                for l in range(GRP):
                    dst = pl.ds(k * GRP + l, 1)
                    pltpu.async_copy(t1_hbm.at[pl.ds(v[l], 1)],
                                     b1.at[dst], s1)
                    pltpu.async_copy(t3_hbm.at[pl.ds(v[l], 1)],
                                     b3.at[dst], s3)
                return 0

            lax.fori_loop(0, CH // GRP, grp_u, 0)
            lax.fori_loop(0, CH // GRP, grp_i, 0)
            pltpu.make_async_copy(out0.at[pl.ds(0, CH)], b0, s0).wait()
            pltpu.make_async_copy(out1.at[pl.ds(0, CH)], b1, s1).wait()
            pltpu.make_async_copy(out2.at[pl.ds(0, CH)], b2, s2).wait()
            pltpu.make_async_copy(out3.at[pl.ds(0, CH)], b3, s3).wait()
            osl = pl.ds(base + k2 * CH, CH)
            pltpu.sync_copy(b0, out0.at[osl])
            pltpu.sync_copy(b1, out1.at[osl])
            pltpu.sync_copy(b2, out2.at[osl])
            pltpu.sync_copy(b3, out3.at[osl])
            return 0

        lax.fori_loop(0, n_chunks, chunk, 0)

    return gather_kernel


def _mlp_body(umf_ref, imf_ref, umlp_ref, imlp_ref,
              w1_ref, b1_ref, w2_ref, b2_ref, wp_ref, bp_ref, out_ref):
    mf = umf_ref[...] * imf_ref[...]
    w1 = w1_ref[...]
    dn = (((1,), (1,)), ((), ()))
    h1 = (lax.dot_general(umlp_ref[...], w1[:, :32], dn,
                          preferred_element_type=jnp.float32)
          + lax.dot_general(imlp_ref[...], w1[:, 32:], dn,
                            preferred_element_type=jnp.float32)
          + b1_ref[...])
    h1 = jnp.maximum(h1, 0.0)
    h2 = lax.dot_general(h1, w2_ref[...], dn,
                         preferred_element_type=jnp.float32) + b2_ref[...]
    h2 = jnp.maximum(h2, 0.0)
    wp = wp_ref[...]
    logit = (lax.dot_general(mf, wp[:, :32], dn,
                             preferred_element_type=jnp.float32)
             + lax.dot_general(h2, wp[:, 32:], dn,
                               preferred_element_type=jnp.float32)
             + bp_ref[...])
    out_ref[...] = jax.nn.sigmoid(logit) * 5.0


@functools.lru_cache(maxsize=None)
def _make_mlp(B, blk, interpret=False):
    n_blocks = B // blk
    return pl.pallas_call(
        _mlp_body,
        grid=(n_blocks,),
        in_specs=[
            pl.BlockSpec((blk, 32), lambda i: (i, 0)),
            pl.BlockSpec((blk, 32), lambda i: (i, 0)),
            pl.BlockSpec((blk, 32), lambda i: (i, 0)),
            pl.BlockSpec((blk, 32), lambda i: (i, 0)),
            pl.BlockSpec((64, 64), lambda i: (0, 0)),
            pl.BlockSpec((1, 64), lambda i: (0, 0)),
            pl.BlockSpec((32, 64), lambda i: (0, 0)),
            pl.BlockSpec((1, 32), lambda i: (0, 0)),
            pl.BlockSpec((1, 64), lambda i: (0, 0)),
            pl.BlockSpec((1, 1), lambda i: (0, 0)),
        ],
        out_specs=pl.BlockSpec((blk, 1), lambda i: (i, 0)),
        out_shape=jax.ShapeDtypeStruct((B, 1), jnp.float32),
        interpret=interpret,
    )


def kernel(user_ids, item_ids, user_mf_emb, item_mf_emb, user_mlp_emb,
           item_mlp_emb, W1, b1, W2, b2, Wp, bp):
    B = user_ids.shape[0]
    gather = _make_gather(B)
    umf, imf, umlp, imlp = gather(user_ids, item_ids, user_mf_emb,
                                  item_mf_emb, user_mlp_emb, item_mlp_emb)
    mlp = _make_mlp(B, 2048)
    return mlp(umf, imf, umlp, imlp,
               W1, b1.reshape(1, -1), W2, b2.reshape(1, -1),
               Wp, bp.reshape(1, 1))
